# baseline (speedup 1.0000x reference)
import jax
import jax.numpy as jnp
from jax.experimental import pallas as pl
from jax.experimental.pallas import tpu as pltpu


def _pos_kernel(w_ref, o_ref, tile_ref, sems):
    h = w = 32
    cy, cx = h // 2, w // 2
    n = h * w
    ij = jax.lax.broadcasted_iota(jnp.int32, (1, n), 1)
    i = ij // w
    j = ij % w
    z = jnp.maximum(jnp.abs(cx - j) + jnp.abs(cy - i) - 1, 0)
    rows = jax.lax.broadcasted_iota(jnp.int32, (h, n), 0)
    onehot = (rows == z).astype(jnp.float32)
    tile_ref[...] = jax.lax.dot_general(
        w_ref[...], onehot,
        dimension_numbers=(((0,), (0,)), ((), ())),
        preferred_element_type=jnp.float32,
    )
    nb = o_ref.shape[0]
    copies = [
        pltpu.make_async_copy(tile_ref, o_ref.at[b], sems.at[b])
        for b in range(nb)
    ]
    for c in copies:
        c.start()
    for c in copies:
        c.wait()


def kernel(x, W):
    b = x.shape[0]
    nf = W.shape[1]
    h, w = x.shape[-2], x.shape[-1]
    n = h * w
    out = pl.pallas_call(
        _pos_kernel,
        in_specs=[pl.BlockSpec(memory_space=pltpu.MemorySpace.VMEM)],
        out_specs=pl.BlockSpec(memory_space=pltpu.MemorySpace.HBM),
        out_shape=jax.ShapeDtypeStruct((b, nf, n), jnp.float32),
        scratch_shapes=[
            pltpu.MemorySpace.VMEM((nf, n), jnp.float32),
            pltpu.SemaphoreType.DMA((b,)),
        ],
        compiler_params=pltpu.CompilerParams(
            disable_bounds_checks=True,
            disable_semaphore_checks=True,
            skip_device_barrier=True,
        ),
    )(W)
    return out.reshape(b, nf, h, w)


# speedup vs baseline: 3.5714x; 3.5714x over previous
import jax
import jax.numpy as jnp
from jax.experimental import pallas as pl
from jax.experimental.pallas import tpu as pltpu


def _pos_kernel(w_ref, o_ref, tile_ref, sems):
    h = w = 32
    cy, cx = h // 2, w // 2
    n = h * w
    nrows = w_ref.shape[0]
    ij = jax.lax.broadcasted_iota(jnp.int32, (n, nrows), 0)
    i = ij // w
    j = ij % w
    z = jnp.maximum(jnp.abs(cx - j) + jnp.abs(cy - i) - 1, 0)
    cols = jax.lax.broadcasted_iota(jnp.int32, (n, nrows), 1)
    onehot = (cols == z).astype(jnp.float32)
    tile_ref[...] = jnp.dot(
        onehot, w_ref[...], preferred_element_type=jnp.float32
    )
    nb = o_ref.shape[0]
    copies = [
        pltpu.make_async_copy(tile_ref, o_ref.at[b], sems.at[b])
        for b in range(nb)
    ]
    for c in copies:
        c.start()
    for c in copies:
        c.wait()


def kernel(x, W):
    b = x.shape[0]
    nf = W.shape[1]
    h, w = x.shape[-2], x.shape[-1]
    n = h * w
    out = pl.pallas_call(
        _pos_kernel,
        in_specs=[pl.BlockSpec(memory_space=pltpu.MemorySpace.VMEM)],
        out_specs=pl.BlockSpec(memory_space=pltpu.MemorySpace.HBM),
        out_shape=jax.ShapeDtypeStruct((b, n, nf), jnp.float32),
        scratch_shapes=[
            pltpu.MemorySpace.VMEM((n, nf), jnp.float32),
            pltpu.SemaphoreType.DMA((b,)),
        ],
    )(W)
    return out.reshape(b, h, w, nf).transpose(0, 3, 1, 2)


# speedup vs baseline: 3.6133x; 1.0117x over previous
import jax
import jax.numpy as jnp
from jax.experimental import pallas as pl
from jax.experimental.pallas import tpu as pltpu


def _pos_kernel(w_ref, o_ref, tile_ref, sems):
    h = w = 32
    cy, cx = h // 2, w // 2
    n = h * w
    nrows = w_ref.shape[0]
    ij = jax.lax.broadcasted_iota(jnp.int32, (n, nrows), 0)
    i = ij // w
    j = ij % w
    z = jnp.maximum(jnp.abs(cx - j) + jnp.abs(cy - i) - 1, 0)
    cols = jax.lax.broadcasted_iota(jnp.int32, (n, nrows), 1)
    onehot = (cols == z).astype(jnp.float32)
    tile = jnp.dot(
        onehot, w_ref[...], preferred_element_type=jnp.float32
    )
    tile_ref[0, :, :] = tile
    tile_ref[1, :, :] = tile
    nb = o_ref.shape[0]
    copies = [
        pltpu.make_async_copy(
            tile_ref, o_ref.at[pl.ds(2 * q, 2)], sems.at[q]
        )
        for q in range(nb // 2)
    ]
    for c in copies:
        c.start()
    for c in copies:
        c.wait()


def kernel(x, W):
    b = x.shape[0]
    nf = W.shape[1]
    h, w = x.shape[-2], x.shape[-1]
    n = h * w
    out = pl.pallas_call(
        _pos_kernel,
        in_specs=[pl.BlockSpec(memory_space=pltpu.MemorySpace.VMEM)],
        out_specs=pl.BlockSpec(memory_space=pltpu.MemorySpace.HBM),
        out_shape=jax.ShapeDtypeStruct((b, n, nf), jnp.float32),
        scratch_shapes=[
            pltpu.MemorySpace.VMEM((2, n, nf), jnp.float32),
            pltpu.SemaphoreType.DMA((b // 2,)),
        ],
    )(W)
    return out.reshape(b, h, w, nf).transpose(0, 3, 1, 2)
